# baseline (speedup 1.0000x reference)
import functools

import jax
import jax.numpy as jnp
from jax import lax
from jax.experimental import pallas as pl
from jax.experimental.pallas import tpu as pltpu
from jax.experimental.pallas import tpu_sc as plsc

N = 10000
E = 320000
D_IN = 128
H = 16
L_EXTRA = 4

NC = 2
NS = 16
NW = NC * NS
PER_W = 10240
E_PAD = NW * PER_W
N_PAD = 10112
ROWS_PER_TILE = N_PAD // NS


def _make_sc_agg(F, chunk, split=False):
    per_w = (E_PAD // NS) if split else PER_W
    nchunk = per_w // chunk
    zrows = ROWS_PER_TILE
    groups = F // H

    def body(h_hbm, src_hbm, dst_hbm, out_hbm,
             srcv0, srcv1, dstv0, dstv1, rows0, rows1, acc,
             gsem0, gsem1, ssem0, ssem1):
        cid = lax.axis_index("c")
        sid = lax.axis_index("s")
        srcv = (srcv0, srcv1)
        dstv = (dstv0, dstv1)
        rows = (rows0, rows1)
        gsem = (gsem0, gsem1)
        ssem = (ssem0, ssem1)

        zr = min(chunk, zrows)

        def _z(i, _):
            for g in range(groups):
                rows0[i, pl.ds(g * H, H)] = jnp.zeros((H,), jnp.float32)
            return 0

        lax.fori_loop(0, zr, _z, 0)
        off = 0
        while off < zrows:
            t = min(zr, zrows - off)
            pltpu.sync_copy(rows0.at[pl.ds(0, t)],
                            acc.at[pl.ds(sid * zrows + off, t)])
            off += t
        plsc.subcore_barrier()

        base = sid * per_w if split else (cid * NS + sid) * per_w

        def load_idx(j):
            b = j % 2
            o = base + j * chunk
            so = cid * E_PAD + o if split else o
            pltpu.sync_copy(src_hbm.at[pl.ds(so, chunk)], srcv[b])
            pltpu.sync_copy(dst_hbm.at[pl.ds(o, chunk)], dstv[b])

        def start_gather(j):
            b = j % 2
            return pltpu.async_copy(h_hbm.at[srcv[b]], rows[b], gsem[b])

        load_idx(0)
        g = [None] * nchunk
        s = [None] * nchunk
        g[0] = start_gather(0)
        for j in range(nchunk):
            b = j % 2
            g[j].wait()
            s[j] = pltpu.async_copy(rows[b], acc.at[dstv[b]], ssem[b], add=True)
            if j + 1 < nchunk:
                if j - 1 >= 0:
                    s[j - 1].wait()
                load_idx(j + 1)
                g[j + 1] = start_gather(j + 1)
        if nchunk >= 2:
            s[nchunk - 2].wait()
        s[nchunk - 1].wait()

        plsc.subcore_barrier()

        if split:
            pltpu.sync_copy(
                acc.at[pl.ds(sid * zrows, zrows)],
                out_hbm.at[pl.ds(sid * zrows, zrows), pl.ds(cid * F, F)],
            )
        else:
            pltpu.sync_copy(
                acc.at[pl.ds(sid * zrows, zrows)],
                out_hbm.at[cid, pl.ds(sid * zrows, zrows)],
            )

    out_shape = (N_PAD, NC * F) if split else (NC, N_PAD, F)
    return functools.partial(
        pl.kernel,
        out_type=jax.ShapeDtypeStruct(out_shape, jnp.float32),
        mesh=plsc.VectorSubcoreMesh(core_axis_name="c", subcore_axis_name="s"),
        scratch_types=[
            pltpu.VMEM((chunk,), jnp.int32),
            pltpu.VMEM((chunk,), jnp.int32),
            pltpu.VMEM((chunk,), jnp.int32),
            pltpu.VMEM((chunk,), jnp.int32),
            pltpu.VMEM((chunk, F), jnp.float32),
            pltpu.VMEM((chunk, F), jnp.float32),
            pltpu.VMEM_SHARED((N_PAD, F), jnp.float32),
            pltpu.SemaphoreType.DMA,
            pltpu.SemaphoreType.DMA,
            pltpu.SemaphoreType.DMA,
            pltpu.SemaphoreType.DMA,
        ],
        compiler_params=pltpu.CompilerParams(use_tc_tiling_on_sc=False),
    )(body)


_sc_agg_wide = _make_sc_agg(D_IN // 2, 640, split=True)
_sc_agg_narrow = _make_sc_agg(H, 1024)


def _mlp_bn(hin, w1, b1, w2, b2, g, beta):
    a = jnp.dot(hin, w1, preferred_element_type=jnp.float32) + b1
    a = jnp.maximum(a, 0.0)
    h = jnp.dot(a, w2, preferred_element_type=jnp.float32) + b2
    h = jnp.maximum(h, 0.0)
    mean = jnp.sum(h, axis=0, keepdims=True) * (1.0 / N)
    c = h - mean
    var = jnp.sum(c * c, axis=0, keepdims=True) * (1.0 / N)
    return c * lax.rsqrt(var + 1e-5) * g + beta


NPK = N // 8


def _fold16(v128):
    out = v128[:, 0:H]
    for g in range(1, 8):
        out = out + v128[:, g * H : (g + 1) * H]
    return out


def _tile128(v16):
    return jnp.concatenate([v16] * 8, axis=1)


def _mlp_bn_packed(hin, bd1, b1t, bd2, b2t, gt, bt):
    a = jnp.dot(hin, bd1, preferred_element_type=jnp.float32) + b1t
    a = jnp.maximum(a, 0.0)
    h = jnp.dot(a, bd2, preferred_element_type=jnp.float32) + b2t
    h = jnp.maximum(h, 0.0)
    m = _tile128(_fold16(jnp.sum(h, axis=0, keepdims=True))) * (1.0 / N)
    c = h - m
    v = _tile128(_fold16(jnp.sum(c * c, axis=0, keepdims=True))) * (1.0 / N)
    return c * lax.rsqrt(v + 1e-5) * gt + bt


def _conv_body(h_ref, p_ref, w1_ref, b1_ref, w2_ref, b2_ref, g_ref, be_ref, o_ref):
    hin = h_ref[...] + p_ref[0, :NPK] + p_ref[1, :NPK]
    o_ref[...] = _mlp_bn_packed(hin, w1_ref[...], b1_ref[...], w2_ref[...],
                                b2_ref[...], g_ref[...], be_ref[...])


def _conv1_body(x_ref, s_ref, w1_ref, b1_ref, w2_ref, b2_ref, g_ref,
                be_ref, o_ref):
    hin = x_ref[...] + s_ref[:NPK]
    o_ref[...] = _mlp_bn_packed(hin, w1_ref[...], b1_ref[...], w2_ref[...],
                                b2_ref[...], g_ref[...], be_ref[...])


def _conv_last_body(h_ref, p_ref, w1_ref, b1_ref, w2_ref, b2_ref, g_ref, be_ref,
                    ones_ref, o_ref):
    hin = h_ref[...] + p_ref[0, :NPK] + p_ref[1, :NPK]
    z = _mlp_bn_packed(hin, w1_ref[...], b1_ref[...], w2_ref[...], b2_ref[...],
                       g_ref[...], be_ref[...])
    e = jnp.exp(z)
    gsum = jnp.dot(e, ones_ref[...], preferred_element_type=jnp.float32)
    o_ref[...] = z - jnp.log(gsum)


_BD_ONES = None


def _conv(hp, parts, w1, b1, w2, b2, g, beta, last=False):
    kr = lambda w: jnp.kron(jnp.eye(8, dtype=jnp.float32), w)
    t = lambda v: jnp.concatenate([v.reshape(1, H)] * 8, axis=1)
    args = [hp, parts, kr(w1), t(b1), kr(w2), t(b2), t(g), t(beta)]
    if last:
        args.append(kr(jnp.ones((H, H), jnp.float32)))
    return pl.pallas_call(
        _conv_last_body if last else _conv_body,
        out_shape=jax.ShapeDtypeStruct((NPK, 8 * H), jnp.float32),
    )(*args)


def kernel(x, edge_index, W1a, b1a, W2a, b2a, ga, ba, Ws1, bs1, Ws2, bs2, gs, bs):
    src = edge_index[0].astype(jnp.int32)
    dst = edge_index[1].astype(jnp.int32)
    pad = E_PAD - E
    ar = jnp.arange(pad, dtype=jnp.int32)
    src = jnp.concatenate([src, ar % N])
    dst = jnp.concatenate([dst, N + ar % (N_PAD - N)])

    r = lambda v: v.reshape(1, H)

    xs = jnp.concatenate([x[:, : D_IN // 2], x[:, D_IN // 2 :]], axis=0)
    src2 = jnp.concatenate([src, src + N])
    p = _sc_agg_wide(xs, src2, dst)
    s_r = p.reshape(N_PAD // 8, 8 * D_IN)
    x_r = x.reshape(NPK, 8 * D_IN)
    kr = lambda w: jnp.kron(jnp.eye(8, dtype=jnp.float32), w)
    t = lambda v: jnp.concatenate([v.reshape(1, H)] * 8, axis=1)
    hp = pl.pallas_call(
        _conv1_body,
        out_shape=jax.ShapeDtypeStruct((NPK, 8 * H), jnp.float32),
    )(x_r, s_r, kr(W1a), t(b1a), kr(W2a), t(b2a), t(ga), t(ba))
    for i in range(L_EXTRA):
        table = hp.reshape(N, H)
        parts = _sc_agg_narrow(table, src, dst).reshape(2, N_PAD // 8, 8 * H)
        hp = _conv(hp, parts, Ws1[i], bs1[i], Ws2[i], bs2[i], gs[i],
                   bs[i], last=(i == L_EXTRA - 1))
    return hp.reshape(N, H)


# speedup vs baseline: 1.1036x; 1.1036x over previous
import functools

import jax
import jax.numpy as jnp
from jax import lax
from jax.experimental import pallas as pl
from jax.experimental.pallas import tpu as pltpu
from jax.experimental.pallas import tpu_sc as plsc

N = 10000
E = 320000
D_IN = 128
H = 16
L_EXTRA = 4

NC = 2
NS = 16
NW = NC * NS
PER_W = 10240
E_PAD = NW * PER_W
N_PAD = 10112
ROWS_PER_TILE = N_PAD // NS


def _make_sc_agg(F, chunk, split=False):
    per_w = (E_PAD // NS) if split else PER_W
    nchunk = per_w // chunk
    zrows = ROWS_PER_TILE
    groups = F // H

    def body(h_hbm, src_hbm, dst_hbm, out_hbm,
             srcv0, srcv1, dstv0, dstv1, rows0, rows1, acc,
             gsem0, gsem1, ssem0, ssem1):
        cid = lax.axis_index("c")
        sid = lax.axis_index("s")
        srcv = (srcv0, srcv1)
        dstv = (dstv0, dstv1)
        rows = (rows0, rows1)
        gsem = (gsem0, gsem1)
        ssem = (ssem0, ssem1)

        zr = min(chunk, zrows)

        def _z(i, _):
            for g in range(groups):
                rows0[i, pl.ds(g * H, H)] = jnp.zeros((H,), jnp.float32)
            return 0

        lax.fori_loop(0, zr, _z, 0)
        off = 0
        while off < zrows:
            t = min(zr, zrows - off)
            pltpu.sync_copy(rows0.at[pl.ds(0, t)],
                            acc.at[pl.ds(sid * zrows + off, t)])
            off += t
        plsc.subcore_barrier()

        base = sid * per_w if split else (cid * NS + sid) * per_w

        def load_idx(j):
            b = j % 2
            o = base + j * chunk
            so = cid * E_PAD + o if split else o
            pltpu.sync_copy(src_hbm.at[pl.ds(so, chunk)], srcv[b])
            pltpu.sync_copy(dst_hbm.at[pl.ds(o, chunk)], dstv[b])

        def start_gather(j):
            b = j % 2
            return pltpu.async_copy(h_hbm.at[srcv[b]], rows[b], gsem[b])

        load_idx(0)
        g = [None] * nchunk
        s = [None] * nchunk
        g[0] = start_gather(0)
        for j in range(nchunk):
            b = j % 2
            g[j].wait()
            s[j] = pltpu.async_copy(rows[b], acc.at[dstv[b]], ssem[b], add=True)
            if j + 1 < nchunk:
                if j - 1 >= 0:
                    s[j - 1].wait()
                load_idx(j + 1)
                g[j + 1] = start_gather(j + 1)
        if nchunk >= 2:
            s[nchunk - 2].wait()
        s[nchunk - 1].wait()

        plsc.subcore_barrier()

        if split:
            pltpu.sync_copy(
                acc.at[pl.ds(sid * zrows, zrows)],
                out_hbm.at[pl.ds(sid * zrows, zrows), pl.ds(cid * F, F)],
            )
        else:
            pltpu.sync_copy(
                acc.at[pl.ds(sid * zrows, zrows)],
                out_hbm.at[cid, pl.ds(sid * zrows, zrows)],
            )

    out_shape = (N_PAD, NC * F) if split else (NC, N_PAD, F)
    return functools.partial(
        pl.kernel,
        out_type=jax.ShapeDtypeStruct(out_shape, jnp.float32),
        mesh=plsc.VectorSubcoreMesh(core_axis_name="c", subcore_axis_name="s"),
        scratch_types=[
            pltpu.VMEM((chunk,), jnp.int32),
            pltpu.VMEM((chunk,), jnp.int32),
            pltpu.VMEM((chunk,), jnp.int32),
            pltpu.VMEM((chunk,), jnp.int32),
            pltpu.VMEM((chunk, F), jnp.float32),
            pltpu.VMEM((chunk, F), jnp.float32),
            pltpu.VMEM_SHARED((N_PAD, F), jnp.float32),
            pltpu.SemaphoreType.DMA,
            pltpu.SemaphoreType.DMA,
            pltpu.SemaphoreType.DMA,
            pltpu.SemaphoreType.DMA,
        ],
        compiler_params=pltpu.CompilerParams(use_tc_tiling_on_sc=False),
    )(body)


_sc_agg_wide = _make_sc_agg(D_IN // 2, 640, split=True)
_sc_agg_narrow = _make_sc_agg(H, 2560)


def _mlp_bn(hin, w1, b1, w2, b2, g, beta):
    a = jnp.dot(hin, w1, preferred_element_type=jnp.float32) + b1
    a = jnp.maximum(a, 0.0)
    h = jnp.dot(a, w2, preferred_element_type=jnp.float32) + b2
    h = jnp.maximum(h, 0.0)
    mean = jnp.sum(h, axis=0, keepdims=True) * (1.0 / N)
    c = h - mean
    var = jnp.sum(c * c, axis=0, keepdims=True) * (1.0 / N)
    return c * lax.rsqrt(var + 1e-5) * g + beta


NPK = N // 8


def _fold16(v128):
    out = v128[:, 0:H]
    for g in range(1, 8):
        out = out + v128[:, g * H : (g + 1) * H]
    return out


def _tile128(v16):
    return jnp.concatenate([v16] * 8, axis=1)


def _mlp_bn_packed(hin, bd1, b1t, bd2, b2t, gt, bt):
    a = jnp.dot(hin, bd1, preferred_element_type=jnp.float32) + b1t
    a = jnp.maximum(a, 0.0)
    h = jnp.dot(a, bd2, preferred_element_type=jnp.float32) + b2t
    h = jnp.maximum(h, 0.0)
    m = _tile128(_fold16(jnp.sum(h, axis=0, keepdims=True))) * (1.0 / N)
    c = h - m
    v = _tile128(_fold16(jnp.sum(c * c, axis=0, keepdims=True))) * (1.0 / N)
    return c * lax.rsqrt(v + 1e-5) * gt + bt


def _conv_body(h_ref, p_ref, w1_ref, b1_ref, w2_ref, b2_ref, g_ref, be_ref, o_ref):
    hin = h_ref[...] + p_ref[0, :NPK] + p_ref[1, :NPK]
    o_ref[...] = _mlp_bn_packed(hin, w1_ref[...], b1_ref[...], w2_ref[...],
                                b2_ref[...], g_ref[...], be_ref[...])


def _conv1_body(x_ref, s_ref, w1_ref, b1_ref, w2_ref, b2_ref, g_ref,
                be_ref, o_ref):
    hin = x_ref[...] + s_ref[:NPK]
    o_ref[...] = _mlp_bn_packed(hin, w1_ref[...], b1_ref[...], w2_ref[...],
                                b2_ref[...], g_ref[...], be_ref[...])


def _conv_last_body(h_ref, p_ref, w1_ref, b1_ref, w2_ref, b2_ref, g_ref, be_ref,
                    ones_ref, o_ref):
    hin = h_ref[...] + p_ref[0, :NPK] + p_ref[1, :NPK]
    z = _mlp_bn_packed(hin, w1_ref[...], b1_ref[...], w2_ref[...], b2_ref[...],
                       g_ref[...], be_ref[...])
    e = jnp.exp(z)
    gsum = jnp.dot(e, ones_ref[...], preferred_element_type=jnp.float32)
    o_ref[...] = z - jnp.log(gsum)


_BD_ONES = None


def _conv(hp, parts, w1, b1, w2, b2, g, beta, last=False):
    kr = lambda w: jnp.kron(jnp.eye(8, dtype=jnp.float32), w)
    t = lambda v: jnp.concatenate([v.reshape(1, H)] * 8, axis=1)
    args = [hp, parts, kr(w1), t(b1), kr(w2), t(b2), t(g), t(beta)]
    if last:
        args.append(kr(jnp.ones((H, H), jnp.float32)))
    return pl.pallas_call(
        _conv_last_body if last else _conv_body,
        out_shape=jax.ShapeDtypeStruct((NPK, 8 * H), jnp.float32),
    )(*args)


def kernel(x, edge_index, W1a, b1a, W2a, b2a, ga, ba, Ws1, bs1, Ws2, bs2, gs, bs):
    src = edge_index[0].astype(jnp.int32)
    dst = edge_index[1].astype(jnp.int32)
    pad = E_PAD - E
    ar = jnp.arange(pad, dtype=jnp.int32)
    src = jnp.concatenate([src, ar % N])
    dst = jnp.concatenate([dst, N + ar % (N_PAD - N)])

    r = lambda v: v.reshape(1, H)

    xs = jnp.concatenate([x[:, : D_IN // 2], x[:, D_IN // 2 :]], axis=0)
    src2 = jnp.concatenate([src, src + N])
    p = _sc_agg_wide(xs, src2, dst)
    s_r = p.reshape(N_PAD // 8, 8 * D_IN)
    x_r = x.reshape(NPK, 8 * D_IN)
    kr = lambda w: jnp.kron(jnp.eye(8, dtype=jnp.float32), w)
    t = lambda v: jnp.concatenate([v.reshape(1, H)] * 8, axis=1)
    hp = pl.pallas_call(
        _conv1_body,
        out_shape=jax.ShapeDtypeStruct((NPK, 8 * H), jnp.float32),
    )(x_r, s_r, kr(W1a), t(b1a), kr(W2a), t(b2a), t(ga), t(ba))
    for i in range(L_EXTRA):
        table = hp.reshape(N, H)
        parts = _sc_agg_narrow(table, src, dst).reshape(2, N_PAD // 8, 8 * H)
        hp = _conv(hp, parts, Ws1[i], bs1[i], Ws2[i], bs2[i], gs[i],
                   bs[i], last=(i == L_EXTRA - 1))
    return hp.reshape(N, H)
